# Initial kernel scaffold; baseline (speedup 1.0000x reference)
import jax
import jax.numpy as jnp
from jax.experimental import pallas as pl


def kernel(qk_dots, rel_bias_table):
    raise NotImplementedError("write your pallas kernel here")


# speedup vs baseline: 31.9345x; 31.9345x over previous
import functools

import jax
import jax.numpy as jnp
from jax.experimental import pallas as pl
from jax.experimental.pallas import tpu as pltpu

_HEADS = 12
_NB = 32
_SCALE = 0.125
_BI = 256
_BJ = 256
_THRESH = (8, 12, 16, 23, 32, 46, 64, 91)


def _bucket_from_rel(rel):
    na = jnp.abs(rel)
    small = na < 8
    vil = jnp.full(rel.shape, 7, jnp.int32)
    for t in _THRESH:
        vil = vil + (na >= t).astype(jnp.int32)
    bk = jnp.where(small, na, vil)
    return bk + jnp.where(rel > 0, 16, 0).astype(jnp.int32)


def _body(qk_ref, tbl_ref, out_ref, d8_ref):
    h = pl.program_id(0)
    ib = pl.program_id(1)
    jb = pl.program_id(2)

    @pl.when(jnp.logical_and(ib == 0, jb == 0))
    def _build_d8():
        t_idx = jax.lax.broadcasted_iota(jnp.int32, (8, 1024), 1)
        s_idx = jax.lax.broadcasted_iota(jnp.int32, (8, 1024), 0)
        rel = t_idx - s_idx - 512
        bk = _bucket_from_rel(rel)
        acc = jnp.zeros((8, 1024), jnp.float32)
        for b in range(_NB):
            acc = jnp.where(bk == b, tbl_ref[b, h], acc)
        d8_ref[...] = acc * _SCALE

    d = jb - ib
    c_lo = tbl_ref[15, h] * _SCALE
    c_up = tbl_ref[31, h] * _SCALE

    @pl.when(d < -1)
    def _lo():
        out_ref[...] = qk_ref[...] + c_lo

    @pl.when(d > 1)
    def _up():
        out_ref[...] = qk_ref[...] + c_up

    for dd in (-1, 0, 1):
        @pl.when(d == dd)
        def _band(dd=dd):
            for q in range(_BI // 8):
                off = 256 * dd + 512 - 8 * q
                out_ref[0, 0, 8 * q:8 * q + 8, :] = (
                    qk_ref[0, 0, 8 * q:8 * q + 8, :]
                    + d8_ref[:, off:off + _BJ]
                )


@jax.jit
def kernel(qk_dots, rel_bias_table):
    i = qk_dots.shape[-2]
    j = qk_dots.shape[-1]
    grid = (_HEADS, i // _BI, j // _BJ)
    return pl.pallas_call(
        _body,
        grid=grid,
        in_specs=[
            pl.BlockSpec((1, 1, _BI, _BJ), lambda h, ib, jb: (0, h, ib, jb)),
            pl.BlockSpec(memory_space=pltpu.SMEM),
        ],
        out_specs=pl.BlockSpec((1, 1, _BI, _BJ), lambda h, ib, jb: (0, h, ib, jb)),
        out_shape=jax.ShapeDtypeStruct(qk_dots.shape, qk_dots.dtype),
        scratch_shapes=[pltpu.VMEM((8, 1024), jnp.float32)],
    )(qk_dots, rel_bias_table)


# speedup vs baseline: 101.7888x; 3.1874x over previous
import jax
import jax.numpy as jnp
from jax.experimental import pallas as pl
from jax.experimental.pallas import tpu as pltpu

_HEADS = 12
_NB = 32
_SCALE = 0.125
_BI = 256
_CH = 256
_THRESH = (8, 12, 16, 23, 32, 46, 64, 91)


def _bucket_from_rel(rel):
    na = jnp.abs(rel)
    small = na < 8
    vil = jnp.full(rel.shape, 7, jnp.int32)
    for t in _THRESH:
        vil = vil + (na >= t).astype(jnp.int32)
    bk = jnp.where(small, na, vil)
    return bk + jnp.where(rel > 0, 16, 0).astype(jnp.int32)


def _body(qk_ref, tbl_ref, out_ref, d8_ref):
    h = pl.program_id(0)
    ib = pl.program_id(1)
    ncols = out_ref.shape[-1]
    nchunks = ncols // _CH

    @pl.when(ib == 0)
    def _build_d8():
        t_idx = jax.lax.broadcasted_iota(jnp.int32, (8, 1024), 1)
        s_idx = jax.lax.broadcasted_iota(jnp.int32, (8, 1024), 0)
        rel = t_idx - s_idx - 512
        bk = _bucket_from_rel(rel)
        acc = jnp.zeros((8, 1024), jnp.float32)
        for b in range(_NB):
            acc = jnp.where(bk == b, tbl_ref[b, h], acc)
        d8_ref[...] = acc * _SCALE

    c_lo = tbl_ref[15, h] * _SCALE
    c_up = tbl_ref[31, h] * _SCALE
    i0 = ib * _BI

    col = jax.lax.broadcasted_iota(jnp.int32, (_BI, ncols), 1)
    base = jnp.where(col < i0, c_lo, c_up)
    out_ref[0, 0, :, :] = qk_ref[0, 0, :, :] + base

    for cc in range(nchunks):
        for dd in (-1, 0, 1):
            if not 0 <= cc - dd < nchunks:
                continue

            @pl.when(ib == cc - dd)
            def _band(cc=cc, dd=dd):
                for q in range(_BI // 8):
                    off = 256 * dd + 512 - 8 * q
                    out_ref[0, 0, 8 * q:8 * q + 8, _CH * cc:_CH * (cc + 1)] = (
                        qk_ref[0, 0, 8 * q:8 * q + 8, _CH * cc:_CH * (cc + 1)]
                        + d8_ref[:, off:off + _CH]
                    )


@jax.jit
def kernel(qk_dots, rel_bias_table):
    i = qk_dots.shape[-2]
    j = qk_dots.shape[-1]
    grid = (_HEADS, i // _BI)
    return pl.pallas_call(
        _body,
        grid=grid,
        in_specs=[
            pl.BlockSpec((1, 1, _BI, j), lambda h, ib: (0, h, ib, 0)),
            pl.BlockSpec(memory_space=pltpu.SMEM),
        ],
        out_specs=pl.BlockSpec((1, 1, _BI, j), lambda h, ib: (0, h, ib, 0)),
        out_shape=jax.ShapeDtypeStruct(qk_dots.shape, qk_dots.dtype),
        scratch_shapes=[pltpu.VMEM((8, 1024), jnp.float32)],
    )(qk_dots, rel_bias_table)


# speedup vs baseline: 121.3679x; 1.1923x over previous
import jax
import jax.numpy as jnp
from jax.experimental import pallas as pl
from jax.experimental.pallas import tpu as pltpu

_HEADS = 12
_NB = 32
_SCALE = 0.125
_BI = 512
_CH = 256
_R = _BI // _CH
_C0 = _BI + 512
_DW = 2048
_THRESH = (8, 12, 16, 23, 32, 46, 64, 91)


def _bucket_from_rel(rel):
    na = jnp.abs(rel)
    small = na < 8
    vil = jnp.full(rel.shape, 7, jnp.int32)
    for t in _THRESH:
        vil = vil + (na >= t).astype(jnp.int32)
    bk = jnp.where(small, na, vil)
    return bk + jnp.where(rel > 0, 16, 0).astype(jnp.int32)


def _body(qk_ref, tbl_ref, out_ref, d8_ref):
    h = pl.program_id(0)
    ib = pl.program_id(1)
    ncols = out_ref.shape[-1]
    nchunks = ncols // _CH

    @pl.when(ib == 0)
    def _build_d8():
        t_idx = jax.lax.broadcasted_iota(jnp.int32, (8, _DW), 1)
        s_idx = jax.lax.broadcasted_iota(jnp.int32, (8, _DW), 0)
        rel = t_idx - s_idx - _C0
        bk = _bucket_from_rel(rel)
        acc = jnp.zeros((8, _DW), jnp.float32)
        for b in range(_NB):
            acc = jnp.where(bk == b, tbl_ref[b, h], acc)
        d8_ref[...] = acc * _SCALE

    c_lo = tbl_ref[15, h] * _SCALE
    c_up = tbl_ref[31, h] * _SCALE
    i0 = ib * _BI

    col = jax.lax.broadcasted_iota(jnp.int32, (_BI, ncols), 1)
    base = jnp.where(col < i0, c_lo, c_up)
    out_ref[0, 0, :, :] = qk_ref[0, 0, :, :] + base

    for cc in range(nchunks):
        for dd in range(-1, _R + 1):
            if (cc - dd) % _R != 0:
                continue
            ibv = (cc - dd) // _R
            if not 0 <= ibv < pl.num_programs(1):
                continue

            @pl.when(ib == ibv)
            def _band(cc=cc, dd=dd):
                for q in range(_BI // 8):
                    off = _CH * dd + _C0 - 8 * q
                    out_ref[0, 0, 8 * q:8 * q + 8, _CH * cc:_CH * (cc + 1)] = (
                        qk_ref[0, 0, 8 * q:8 * q + 8, _CH * cc:_CH * (cc + 1)]
                        + d8_ref[:, off:off + _CH]
                    )


@jax.jit
def kernel(qk_dots, rel_bias_table):
    i = qk_dots.shape[-2]
    j = qk_dots.shape[-1]
    grid = (_HEADS, i // _BI)
    return pl.pallas_call(
        _body,
        grid=grid,
        in_specs=[
            pl.BlockSpec((1, 1, _BI, j), lambda h, ib: (0, h, ib, 0)),
            pl.BlockSpec(memory_space=pltpu.SMEM),
        ],
        out_specs=pl.BlockSpec((1, 1, _BI, j), lambda h, ib: (0, h, ib, 0)),
        out_shape=jax.ShapeDtypeStruct(qk_dots.shape, qk_dots.dtype),
        scratch_shapes=[pltpu.VMEM((8, _DW), jnp.float32)],
    )(qk_dots, rel_bias_table)


# speedup vs baseline: 124.7483x; 1.0279x over previous
import jax
import jax.numpy as jnp
from jax.experimental import pallas as pl
from jax.experimental.pallas import tpu as pltpu

_HEADS = 12
_NB = 32
_SCALE = 0.125
_BI = 1024
_CH = 256
_R = _BI // _CH
_C0 = _BI + 512
_DW = _C0 + _BI + 512
_THRESH = (8, 12, 16, 23, 32, 46, 64, 91)


def _bucket_from_rel(rel):
    na = jnp.abs(rel)
    small = na < 8
    vil = jnp.full(rel.shape, 7, jnp.int32)
    for t in _THRESH:
        vil = vil + (na >= t).astype(jnp.int32)
    bk = jnp.where(small, na, vil)
    return bk + jnp.where(rel > 0, 16, 0).astype(jnp.int32)


def _body(qk_ref, tbl_ref, out_ref, d8_ref):
    h = pl.program_id(0)
    ib = pl.program_id(1)
    ncols = out_ref.shape[-1]
    nchunks = ncols // _CH

    @pl.when(ib == 0)
    def _build_d8():
        t_idx = jax.lax.broadcasted_iota(jnp.int32, (8, _DW), 1)
        s_idx = jax.lax.broadcasted_iota(jnp.int32, (8, _DW), 0)
        rel = t_idx - s_idx - _C0
        bk = _bucket_from_rel(rel)
        acc = jnp.zeros((8, _DW), jnp.float32)
        for b in range(_NB):
            acc = jnp.where(bk == b, tbl_ref[b, h], acc)
        d8_ref[...] = acc * _SCALE

    c_lo = tbl_ref[15, h] * _SCALE
    c_up = tbl_ref[31, h] * _SCALE
    i0 = ib * _BI

    col = jax.lax.broadcasted_iota(jnp.int32, (_BI, ncols), 1)
    base = jnp.where(col < i0, c_lo, c_up)
    out_ref[0, 0, :, :] = qk_ref[0, 0, :, :] + base

    for cc in range(nchunks):
        for dd in range(-1, _R + 1):
            if (cc - dd) % _R != 0:
                continue
            ibv = (cc - dd) // _R
            if not 0 <= ibv < pl.num_programs(1):
                continue

            @pl.when(ib == ibv)
            def _band(cc=cc, dd=dd):
                for q in range(_BI // 8):
                    off = _CH * dd + _C0 - 8 * q
                    out_ref[0, 0, 8 * q:8 * q + 8, _CH * cc:_CH * (cc + 1)] = (
                        qk_ref[0, 0, 8 * q:8 * q + 8, _CH * cc:_CH * (cc + 1)]
                        + d8_ref[:, off:off + _CH]
                    )


@jax.jit
def kernel(qk_dots, rel_bias_table):
    i = qk_dots.shape[-2]
    j = qk_dots.shape[-1]
    grid = (_HEADS, i // _BI)
    return pl.pallas_call(
        _body,
        grid=grid,
        in_specs=[
            pl.BlockSpec((1, 1, _BI, j), lambda h, ib: (0, h, ib, 0)),
            pl.BlockSpec(memory_space=pltpu.SMEM),
        ],
        out_specs=pl.BlockSpec((1, 1, _BI, j), lambda h, ib: (0, h, ib, 0)),
        out_shape=jax.ShapeDtypeStruct(qk_dots.shape, qk_dots.dtype),
        scratch_shapes=[pltpu.VMEM((8, _DW), jnp.float32)],
    )(qk_dots, rel_bias_table)
